# Initial kernel scaffold; baseline (speedup 1.0000x reference)
import jax
import jax.numpy as jnp
from jax.experimental import pallas as pl


def kernel(x, cond, params, edge_index, batch):
    raise NotImplementedError("write your pallas kernel here")


# speedup vs baseline: 9.5891x; 9.5891x over previous
import functools
import math

import numpy as np
import jax
import jax.numpy as jnp
from jax import lax
from jax.experimental import pallas as pl
from jax.experimental.pallas import tpu as pltpu
from jax.experimental.pallas import tpu_sc as plsc

N = 10000
E = 320000
H = 128
LAT = 32
HD = 256
NMAX = 50
DC = 128
NG = 200
AH = NMAX * (NMAX - 1) // 2
AHP = 1280
ADJF = NMAX * NMAX

NC, NS = 2, 16
NW = NC * NS
EPT = E // NW
EB = 125
KB = EPT // EB
CHUNK = 16
NCHUNK = KB // CHUNK
RPT = 624
TOFF = NS * RPT
TAIL = N - TOFF

BR = 1000
NBLK = N // BR

_BNS = 1.0 / math.sqrt(1.0 + 1e-5)

_IU = np.triu_indices(NMAX, 1)
_PFULL = np.zeros((AHP, ADJF), np.float32)
_PFULL[np.arange(AH), _IU[0] * NMAX + _IU[1]] = 1.0
_PFULL[np.arange(AH), _IU[1] * NMAX + _IU[0]] = 1.0


def _leaky(t):
    return jnp.where(t > 0, t, 0.2 * t)


def _sc_scatter_body(h_hbm, z_hbm, src_hbm, dst_hbm, out_hbm,
                     src_v, dst_v, buf0, buf1, agg, semA, semB):
    cid = lax.axis_index("c")
    sid = lax.axis_index("s")
    wid = cid * NS + sid
    row0 = sid * RPT

    @pl.when(cid == 0)
    def _():
        pltpu.sync_copy(h_hbm.at[pl.ds(row0, RPT)], agg.at[pl.ds(row0, RPT)])

        @pl.when(sid == NS - 1)
        def _():
            pltpu.sync_copy(h_hbm.at[pl.ds(TOFF, TAIL)], agg.at[pl.ds(TOFF, TAIL)])

    @pl.when(cid != 0)
    def _():
        pltpu.sync_copy(z_hbm.at[pl.ds(row0, RPT)], agg.at[pl.ds(row0, RPT)])

        @pl.when(sid == NS - 1)
        def _():
            pltpu.sync_copy(z_hbm.at[pl.ds(TOFF, TAIL)], agg.at[pl.ds(TOFF, TAIL)])

    plsc.subcore_barrier()

    @pl.loop(0, NCHUNK)
    def _(ci):
        cbase = wid * KB + ci * CHUNK
        pltpu.sync_copy(src_hbm.at[pl.ds(cbase, CHUNK)], src_v)
        pltpu.sync_copy(dst_hbm.at[pl.ds(cbase, CHUNK)], dst_v)
        pltpu.async_copy(h_hbm.at[src_v.at[0]], buf0, semA)
        pltpu.async_copy(h_hbm.at[src_v.at[1]], buf1, semB)

        @pl.loop(0, CHUNK, step=2)
        def _(i):
            pltpu.make_async_copy(h_hbm.at[src_v.at[i]], buf0, semA).wait()
            pltpu.sync_copy(buf0, agg.at[dst_v.at[i]], add=True)

            @pl.when(i + 2 < CHUNK)
            def _():
                pltpu.async_copy(h_hbm.at[src_v.at[i + 2]], buf0, semA)

            pltpu.make_async_copy(h_hbm.at[src_v.at[i + 1]], buf1, semB).wait()
            pltpu.sync_copy(buf1, agg.at[dst_v.at[i + 1]], add=True)

            @pl.when(i + 3 < CHUNK)
            def _():
                pltpu.async_copy(h_hbm.at[src_v.at[i + 3]], buf1, semB)

    plsc.subcore_barrier()

    pltpu.sync_copy(agg.at[pl.ds(row0, RPT)], out_hbm.at[cid, pl.ds(row0, RPT)])

    @pl.when(sid == NS - 1)
    def _():
        pltpu.sync_copy(agg.at[pl.ds(TOFF, TAIL)], out_hbm.at[cid, pl.ds(TOFF, TAIL)])


@functools.lru_cache(maxsize=1)
def _sc_scatter_kernel():
    return pl.kernel(
        _sc_scatter_body,
        out_type=jax.ShapeDtypeStruct((2, N, H), jnp.float32),
        mesh=plsc.VectorSubcoreMesh(core_axis_name="c", subcore_axis_name="s",
                                    num_cores=NC, num_subcores=NS),
        scratch_types=[
            pltpu.VMEM((CHUNK, EB), jnp.int32),
            pltpu.VMEM((CHUNK, EB), jnp.int32),
            pltpu.VMEM((EB, H), jnp.float32),
            pltpu.VMEM((EB, H), jnp.float32),
            pltpu.VMEM_SHARED((N, H), jnp.float32),
            pltpu.SemaphoreType.DMA,
            pltpu.SemaphoreType.DMA,
        ],
    )


def _sc_scatter(h, zeros, src2, dst2):
    return _sc_scatter_kernel()(h, zeros, src2, dst2)


def _mlp_body(p0_ref, p1_ref, W_ref, b_ref, s_ref, be_ref, W2_ref, b2_ref, o_ref):
    a = p0_ref[...] + p1_ref[...]
    t = jnp.dot(a, W_ref[...], preferred_element_type=jnp.float32) + b_ref[...]
    t = _leaky(t)
    t = t * s_ref[...] + be_ref[...]
    t = jnp.dot(t, W2_ref[...], preferred_element_type=jnp.float32) + b2_ref[...]
    o_ref[...] = _leaky(t)


def _mlp(p0, p1, W, b, s, be, W2, b2):
    full = lambda shp: pl.BlockSpec(shp, lambda i: (0,) * len(shp))
    return pl.pallas_call(
        _mlp_body,
        grid=(NBLK,),
        in_specs=[
            pl.BlockSpec((BR, H), lambda i: (i, 0)),
            pl.BlockSpec((BR, H), lambda i: (i, 0)),
            full((H, H)), full((1, H)), full((1, H)), full((1, H)),
            full((H, H)), full((1, H)),
        ],
        out_specs=pl.BlockSpec((BR, H), lambda i: (i, 0)),
        out_shape=jax.ShapeDtypeStruct((N, H), jnp.float32),
    )(p0, p1, W, b, s, be, W2, b2)


def _mlp_pool_body(p0_ref, p1_ref, batch_ref, W_ref, b_ref, s_ref, be_ref,
                   W2_ref, b2_ref, o_ref):
    a = p0_ref[...] + p1_ref[...]
    t = jnp.dot(a, W_ref[...], preferred_element_type=jnp.float32) + b_ref[...]
    t = _leaky(t)
    t = t * s_ref[...] + be_ref[...]
    t = jnp.dot(t, W2_ref[...], preferred_element_type=jnp.float32) + b2_ref[...]
    t = _leaky(t)
    bb = batch_ref[0]
    onehot = (lax.broadcasted_iota(jnp.int32, (NG, BR), 0) == bb)
    contrib = jnp.dot(onehot.astype(jnp.float32), t,
                      preferred_element_type=jnp.float32)

    @pl.when(pl.program_id(0) == 0)
    def _():
        o_ref[...] = contrib

    @pl.when(pl.program_id(0) != 0)
    def _():
        o_ref[...] += contrib


def _mlp_pool(p0, p1, batch3, W, b, s, be, W2, b2):
    full = lambda shp: pl.BlockSpec(shp, lambda i: (0,) * len(shp))
    return pl.pallas_call(
        _mlp_pool_body,
        grid=(NBLK,),
        in_specs=[
            pl.BlockSpec((BR, H), lambda i: (i, 0)),
            pl.BlockSpec((BR, H), lambda i: (i, 0)),
            pl.BlockSpec((1, 1, BR), lambda i: (i, 0, 0)),
            full((H, H)), full((1, H)), full((1, H)), full((1, H)),
            full((H, H)), full((1, H)),
        ],
        out_specs=pl.BlockSpec((NG, H), lambda i: (0, 0)),
        out_shape=jax.ShapeDtypeStruct((NG, H), jnp.float32),
    )(p0, p1, batch3, W, b, s, be, W2, b2)


def _decoder_body(g_ref, cond_ref, es_ref, eb_ref, fcW_ref, fcb_ref,
                  muW_ref, mub_ref, c0W_ref, c0b_ref, c1W_ref, c1b_ref,
                  d0Wz_ref, d0Wc_ref, d0b_ref, s0_ref, b0_ref,
                  d1Wh_ref, d1Wc_ref, d1b_ref, s1_ref, b1_ref,
                  Wd_ref, gdt_ref, P_ref, o_ref):
    f32 = jnp.float32
    dot = lambda a, b: jnp.dot(a, b, preferred_element_type=f32)
    gb = g_ref[...] * es_ref[...] + eb_ref[...]
    gf = dot(gb, fcW_ref[...]) + fcb_ref[...]
    z = dot(gf, muW_ref[...]) + mub_ref[...]
    c = jnp.maximum(dot(cond_ref[...], c0W_ref[...]) + c0b_ref[...], 0.0)
    c = dot(c, c1W_ref[...]) + c1b_ref[...]
    h0 = jnp.maximum(dot(z, d0Wz_ref[...]) + dot(c, d0Wc_ref[...])
                     + d0b_ref[...], 0.0)
    h0 = h0 * s0_ref[...] + b0_ref[...]
    h1 = jnp.maximum(dot(h0, d1Wh_ref[...]) + dot(c, d1Wc_ref[...])
                     + d1b_ref[...], 0.0)
    h1 = h1 * s1_ref[...] + b1_ref[...]
    delta = dot(h1, Wd_ref[...]) + gdt_ref[...]
    vals = (delta >= 0).astype(jnp.bfloat16)
    o_ref[...] = dot(vals, P_ref[...])


def _decoder(g, cond, es, eb, fcW, fcb, muW, mub, c0W, c0b, c1W, c1b,
             d0Wz, d0Wc, d0b, s0, b0, d1Wh, d1Wc, d1b, s1, b1, Wd, gdt, P):
    return pl.pallas_call(
        _decoder_body,
        out_shape=jax.ShapeDtypeStruct((NG, ADJF), jnp.float32),
    )(g, cond, es, eb, fcW, fcb, muW, mub, c0W, c0b, c1W, c1b,
      d0Wz, d0Wc, d0b, s0, b0, d1Wh, d1Wc, d1b, s1, b1, Wd, gdt, P)


def kernel(x, cond, params, edge_index, batch):
    p = params
    f32 = jnp.float32
    row = lambda v: v.reshape(1, -1).astype(f32)

    src2 = edge_index[0].reshape(NW * KB, EB)
    dst2 = edge_index[1].reshape(NW * KB, EB)
    zeros = jnp.zeros((N, H), f32)
    batch3 = batch.reshape(NBLK, 1, BR)

    pp = _sc_scatter(x, zeros, src2, dst2)
    h1 = _mlp(pp[0], pp[1], p['c0W'], row(p['c0b']), row(p['c0g'] * _BNS),
              row(p['c0be']), p['c0W2'], row(p['c0b2']))
    pp2 = _sc_scatter(h1, zeros, src2, dst2)
    g = _mlp_pool(pp2[0], pp2[1], batch3, p['c1W'], row(p['c1b']),
                  row(p['c1g'] * _BNS), row(p['c1be']), p['c1W2'], row(p['c1b2']))

    Wd = p['d2W'][:, 0::2] - p['d2W'][:, 1::2]
    Wd = jnp.pad(Wd, ((0, 0), (0, AHP - AH)))
    gn = jax.random.gumbel(jax.random.key(42), (NG, AH, 2), jnp.float32)
    gdiff = gn[:, :, 0] - gn[:, :, 1] + (p['d2b'][0::2] - p['d2b'][1::2])[None, :]
    gdt = jnp.pad(gdiff, ((0, 0), (0, AHP - AH)), constant_values=-1e9)
    P = jnp.asarray(_PFULL, jnp.bfloat16)

    adjf = _decoder(
        g, cond, row(p['ebn_g'] * _BNS), row(p['ebn_b']),
        p['fcW'], row(p['fcb']), p['muW'], row(p['mub']),
        p['cm0W'], row(p['cm0b']), p['cm1W'], row(p['cm1b']),
        p['d0W'][:LAT], p['d0W'][LAT:], row(p['d0b']),
        row(p['dbn0_g'] * _BNS), row(p['dbn0_b']),
        p['d1W'][:HD], p['d1W'][HD:], row(p['d1b']),
        row(p['dbn1_g'] * _BNS), row(p['dbn1_b']),
        Wd, gdt, P)
    return adjf.reshape(NG, NMAX, NMAX)
